# baseline (speedup 1.0000x reference)
import functools

import jax
import jax.numpy as jnp
from jax.experimental import pallas as pl
from jax.experimental.pallas import tpu as pltpu

B, S, D_IN, D_OUT = 4, 2048, 1024, 1024
E, R, TOP_K = 8, 8, 2
SCALING = 32 / R

TOKENS = B * S
BLOCK_T = 1024
RP = 128


def _fused_kernel(x_ref, w_ref, b_ref, bcat_ref, rw_ref, rb_ref,
                  expand_ref, out_ref):
    xb = x_ref[...]

    y = jax.lax.dot_general(
        xb, w_ref[...], (((1,), (0,)), ((), ())),
        preferred_element_type=jnp.float32)
    base = y[:, :D_OUT]
    ax = y[:, D_OUT:]

    logits = jax.lax.dot_general(
        rw_ref[...], xb, (((1,), (1,)), ((), ())),
        preferred_element_type=jnp.float32) + rb_ref[...]

    m1 = jnp.max(logits, axis=0, keepdims=True)
    row = jax.lax.broadcasted_iota(jnp.int32, logits.shape, 0)
    is_max = logits == m1
    first_max = jnp.min(jnp.where(is_max, row, E), axis=0, keepdims=True)
    wo_top1 = jnp.where(row == first_max, -jnp.inf, logits)
    kth = jnp.max(wo_top1, axis=0, keepdims=True)
    masked = jnp.where(logits < kth, -1e9, logits)

    e = jnp.exp(masked - m1)
    gates = e / jnp.sum(e, axis=0, keepdims=True)

    gates_p = jax.lax.dot_general(
        gates, expand_ref[...], (((0,), (0,)), ((), ())),
        preferred_element_type=jnp.float32)

    lora = jax.lax.dot_general(
        ax * gates_p, bcat_ref[...], (((1,), (0,)), ((), ())),
        preferred_element_type=jnp.float32)

    out_ref[...] = base + b_ref[...] + lora


@jax.jit
def kernel(x, base_W, base_b, lora_A, lora_B, router_W, router_b):
    x2 = x.reshape(TOKENS, D_IN)
    a_cat = jnp.transpose(lora_A, (1, 0, 2)).reshape(D_IN, E * R)
    w_aug = jnp.concatenate(
        [base_W.T, a_cat,
         jnp.zeros((D_IN, RP - E * R), jnp.float32)], axis=1)
    b_cat = jnp.concatenate(
        [SCALING * lora_B.reshape(E * R, D_OUT),
         jnp.zeros((RP - E * R, D_OUT), jnp.float32)], axis=0)
    bias = base_b.reshape(1, D_OUT)
    rb = router_b.reshape(E, 1)
    col = jax.lax.broadcasted_iota(jnp.int32, (E, RP), 1)
    expand = ((jax.lax.broadcasted_iota(jnp.int32, (E, RP), 0) == col // R)
              & (col < E * R)).astype(jnp.float32)

    grid = (TOKENS // BLOCK_T,)
    out = pl.pallas_call(
        _fused_kernel,
        grid=grid,
        in_specs=[
            pl.BlockSpec((BLOCK_T, D_IN), lambda i: (i, 0)),
            pl.BlockSpec((D_IN, D_OUT + RP), lambda i: (0, 0)),
            pl.BlockSpec((1, D_OUT), lambda i: (0, 0)),
            pl.BlockSpec((RP, D_OUT), lambda i: (0, 0)),
            pl.BlockSpec((E, D_IN), lambda i: (0, 0)),
            pl.BlockSpec((E, 1), lambda i: (0, 0)),
            pl.BlockSpec((E, RP), lambda i: (0, 0)),
        ],
        out_specs=pl.BlockSpec((BLOCK_T, D_OUT), lambda i: (i, 0)),
        out_shape=jax.ShapeDtypeStruct((TOKENS, D_OUT), jnp.float32),
        compiler_params=pltpu.CompilerParams(
            dimension_semantics=("arbitrary",)),
    )(x2, w_aug, bias, b_cat, router_W, rb, expand)
    return out.reshape(B, S, D_OUT)


# speedup vs baseline: 1.1725x; 1.1725x over previous
import functools

import jax
import jax.numpy as jnp
from jax.experimental import pallas as pl
from jax.experimental.pallas import tpu as pltpu

B, S, D_IN, D_OUT = 4, 2048, 1024, 1024
E, R, TOP_K = 8, 8, 2
SCALING = 32 / R

TOKENS = B * S
BLOCK_T = 1024
RP = 128


def _fused_kernel(x_ref, w_ref, b_ref, acat_ref, bcat_ref, rw_ref, rb_ref,
                  expand_ref, out_ref):
    xb = x_ref[...]

    base = jax.lax.dot_general(
        xb, w_ref[...], (((1,), (1,)), ((), ())),
        preferred_element_type=jnp.float32)

    logits = jax.lax.dot_general(
        rw_ref[...], xb, (((1,), (1,)), ((), ())),
        preferred_element_type=jnp.float32) + rb_ref[...]

    m1 = jnp.max(logits, axis=0, keepdims=True)
    row = jax.lax.broadcasted_iota(jnp.int32, logits.shape, 0)
    is_max = logits == m1
    first_max = jnp.min(jnp.where(is_max, row, E), axis=0, keepdims=True)
    wo_top1 = jnp.where(row == first_max, -jnp.inf, logits)
    kth = jnp.max(wo_top1, axis=0, keepdims=True)
    masked = jnp.where(logits < kth, -1e9, logits)

    e = jnp.exp(masked - m1)
    gates = e / jnp.sum(e, axis=0, keepdims=True)

    gates64 = jax.lax.dot_general(
        gates, expand_ref[...], (((0,), (0,)), ((), ())),
        preferred_element_type=jnp.float32)

    ax = jax.lax.dot_general(
        xb, acat_ref[...], (((1,), (0,)), ((), ())),
        preferred_element_type=jnp.float32)
    lora = jax.lax.dot_general(
        ax * gates64, bcat_ref[...], (((1,), (0,)), ((), ())),
        preferred_element_type=jnp.float32)

    out_ref[...] = base + b_ref[...] + lora


@jax.jit
def kernel(x, base_W, base_b, lora_A, lora_B, router_W, router_b):
    x2 = x.reshape(TOKENS, D_IN)
    a_cat = jnp.transpose(lora_A, (1, 0, 2)).reshape(D_IN, E * R)
    b_cat = SCALING * lora_B.reshape(E * R, D_OUT)
    bias = base_b.reshape(1, D_OUT)
    rb = router_b.reshape(E, 1)
    expand = (jax.lax.broadcasted_iota(jnp.int32, (E, E * R), 0)
              == jax.lax.broadcasted_iota(jnp.int32, (E, E * R), 1) // R
              ).astype(jnp.float32)

    grid = (TOKENS // BLOCK_T,)
    out = pl.pallas_call(
        _fused_kernel,
        grid=grid,
        in_specs=[
            pl.BlockSpec((BLOCK_T, D_IN), lambda i: (i, 0)),
            pl.BlockSpec((D_OUT, D_IN), lambda i: (0, 0)),
            pl.BlockSpec((1, D_OUT), lambda i: (0, 0)),
            pl.BlockSpec((D_IN, E * R), lambda i: (0, 0)),
            pl.BlockSpec((E * R, D_OUT), lambda i: (0, 0)),
            pl.BlockSpec((E, D_IN), lambda i: (0, 0)),
            pl.BlockSpec((E, 1), lambda i: (0, 0)),
            pl.BlockSpec((E, E * R), lambda i: (0, 0)),
        ],
        out_specs=pl.BlockSpec((BLOCK_T, D_OUT), lambda i: (i, 0)),
        out_shape=jax.ShapeDtypeStruct((TOKENS, D_OUT), jnp.float32),
        compiler_params=pltpu.CompilerParams(
            dimension_semantics=("arbitrary",)),
    )(x2, base_W, bias, a_cat, b_cat, router_W, rb, expand)
    return out.reshape(B, S, D_OUT)


# speedup vs baseline: 1.2448x; 1.0617x over previous
import functools

import jax
import jax.numpy as jnp
from jax.experimental import pallas as pl
from jax.experimental.pallas import tpu as pltpu

B, S, D_IN, D_OUT = 4, 2048, 1024, 1024
E, R, TOP_K = 8, 8, 2
SCALING = 32 / R

TOKENS = B * S
BLOCK_T = 1024
RP = 128


def _fused_kernel(x_ref, w_ref, b_ref, acat_ref, bcat_ref, rw_ref, rb_ref,
                  out_ref):
    xb = x_ref[...]

    base = jax.lax.dot_general(
        xb, w_ref[...], (((1,), (1,)), ((), ())),
        preferred_element_type=jnp.float32)

    logits = jax.lax.dot_general(
        rw_ref[...], xb, (((1,), (1,)), ((), ())),
        preferred_element_type=jnp.float32) + rb_ref[...]

    m1 = jnp.max(logits, axis=0, keepdims=True)
    row = jax.lax.broadcasted_iota(jnp.int32, logits.shape, 0)
    is_max = logits == m1
    first_max = jnp.min(jnp.where(is_max, row, E), axis=0, keepdims=True)
    wo_top1 = jnp.where(row == first_max, -jnp.inf, logits)
    kth = jnp.max(wo_top1, axis=0, keepdims=True)
    masked = jnp.where(logits < kth, -1e9, logits)

    e = jnp.exp(masked - m1)
    gates = e / jnp.sum(e, axis=0, keepdims=True)

    expand = (jax.lax.broadcasted_iota(jnp.int32, (E, E * R), 0)
              == jax.lax.broadcasted_iota(jnp.int32, (E, E * R), 1) // R
              ).astype(jnp.float32)
    gates64 = jax.lax.dot_general(
        gates, expand, (((0,), (0,)), ((), ())),
        preferred_element_type=jnp.float32)

    ax = jax.lax.dot_general(
        xb, acat_ref[...], (((1,), (0,)), ((), ())),
        preferred_element_type=jnp.float32)
    lora = jax.lax.dot_general(
        (SCALING * ax) * gates64, bcat_ref[...], (((1,), (0,)), ((), ())),
        preferred_element_type=jnp.float32)

    out_ref[...] = base + b_ref[...] + lora


@jax.jit
def kernel(x, base_W, base_b, lora_A, lora_B, router_W, router_b):
    x2 = x.reshape(TOKENS, D_IN)
    a_cat = jnp.transpose(lora_A, (1, 0, 2)).reshape(D_IN, E * R)
    b_cat = lora_B.reshape(E * R, D_OUT)
    bias = base_b.reshape(1, D_OUT)
    rb = router_b.reshape(E, 1)

    grid = (TOKENS // BLOCK_T,)
    out = pl.pallas_call(
        _fused_kernel,
        grid=grid,
        in_specs=[
            pl.BlockSpec((BLOCK_T, D_IN), lambda i: (i, 0)),
            pl.BlockSpec((D_OUT, D_IN), lambda i: (0, 0)),
            pl.BlockSpec((1, D_OUT), lambda i: (0, 0)),
            pl.BlockSpec((D_IN, E * R), lambda i: (0, 0)),
            pl.BlockSpec((E * R, D_OUT), lambda i: (0, 0)),
            pl.BlockSpec((E, D_IN), lambda i: (0, 0)),
            pl.BlockSpec((E, 1), lambda i: (0, 0)),
        ],
        out_specs=pl.BlockSpec((BLOCK_T, D_OUT), lambda i: (i, 0)),
        out_shape=jax.ShapeDtypeStruct((TOKENS, D_OUT), jnp.float32),
        compiler_params=pltpu.CompilerParams(
            dimension_semantics=("arbitrary",)),
    )(x2, base_W, bias, a_cat, b_cat, router_W, rb)
    return out.reshape(B, S, D_OUT)
